# baseline (speedup 1.0000x reference)
import jax
import jax.numpy as jnp
from jax.experimental import pallas as pl
from jax.experimental.pallas import tpu as pltpu


def kernel(x, spatial_emb):
    b, c, h, w = x.shape
    g = spatial_emb.shape[1]
    ch, cw = h // g, w // g
    band = ch * w
    k = 1
    while (k * band) % 128:
        k += 1
    lanes = k * band
    nrg = g // k
    kg = k * g
    rows_total = b * c * nrg

    tab = jnp.transpose(spatial_emb[0], (2, 0, 1)).reshape(c * nrg, kg)
    x2 = x.reshape(rows_total, lanes)

    BR = 512
    nblocks = rows_total // BR
    per_b = c * nrg // BR

    def body(tab_ref, x_ref, o_ref, sel_ref):
        @pl.when(pl.program_id(0) == 0)
        def _():
            l = jax.lax.broadcasted_iota(jnp.int32, (1, lanes), 1)
            code = (l // band) * g + (l % w) // cw
            gg = jax.lax.broadcasted_iota(jnp.int32, (kg, lanes), 0)
            sel_ref[...] = (code == gg).astype(jnp.float32)
        rows = jnp.dot(tab_ref[...], sel_ref[...],
                       preferred_element_type=jnp.float32)
        o_ref[...] = x_ref[...] + rows

    out = pl.pallas_call(
        body,
        grid=(nblocks,),
        in_specs=[
            pl.BlockSpec((BR, kg), lambda i: (i % per_b, 0)),
            pl.BlockSpec((BR, lanes), lambda i: (i, 0)),
        ],
        out_specs=pl.BlockSpec((BR, lanes), lambda i: (i, 0)),
        out_shape=jax.ShapeDtypeStruct((rows_total, lanes), x.dtype),
        scratch_shapes=[pltpu.VMEM((kg, lanes), jnp.float32)],
    )(tab, x2)
    return out.reshape(b, c, h, w)
